# Initial kernel scaffold; baseline (speedup 1.0000x reference)
import jax
import jax.numpy as jnp
from jax.experimental import pallas as pl


def kernel(x, edge_index, W, b):
    raise NotImplementedError("write your pallas kernel here")


# speedup vs baseline: 17.5626x; 17.5626x over previous
import dataclasses
import functools

import jax
import jax.numpy as jnp
from jax import lax
from jax.experimental import pallas as pl
from jax.experimental.pallas import tpu as pltpu
from jax.experimental.pallas import tpu_sc as plsc

_N = 10000
_E = 320000
_D = 128
_C = 64
_NPAD = 10240
_HALF = 32
_RPT = _NPAD // 16
_CHUNK = 80
_NSUB = 16

_DEG_EPT = _E // 32
_DEG_CHUNKS = _DEG_EPT // _CHUNK
_PROP_EPT = _E // _NSUB
_PROP_CHUNKS = _PROP_EPT // _CHUNK

_mesh = plsc.VectorSubcoreMesh(core_axis_name="c", subcore_axis_name="s")

_cp = pltpu.CompilerParams(needs_layout_passes=False,
                           use_tc_tiling_on_sc=False)


def _fill(buf, nvec, val):
    def body(j, _):
        buf[pl.ds(j * 16, 16)] = jnp.full((16,), val, jnp.float32)
        return _
    lax.fori_loop(0, nvec, body, None)


@functools.partial(
    pl.kernel,
    out_type=jax.ShapeDtypeStruct((2, _NPAD), jnp.float32),
    mesh=_mesh,
    scratch_types=[
        pltpu.VMEM((_DEG_CHUNKS, _CHUNK), jnp.int32),
        pltpu.VMEM((_CHUNK,), jnp.float32),
        pltpu.VMEM((_RPT,), jnp.float32),
        pltpu.VMEM_SHARED((_NPAD,), jnp.float32),
    ],
)
def _deg_kernel(dst_hbm, degp_hbm, dst_v, ones_v, buf_v, deg_sp):
    c = lax.axis_index("c")
    s = lax.axis_index("s")
    wid = c * _NSUB + s
    r0 = s * _RPT
    _fill(buf_v, _RPT // 16, jnp.where(c == 0, 1.0, 0.0))
    pltpu.sync_copy(buf_v, deg_sp.at[pl.ds(r0, _RPT)])
    _fill(ones_v, _CHUNK // 16, 1.0)
    pltpu.sync_copy(dst_hbm.at[wid], dst_v)
    plsc.subcore_barrier()

    def edge_loop(i, _):
        pltpu.sync_copy(ones_v, deg_sp.at[dst_v.at[i]], add=True)
        return _
    lax.fori_loop(0, _DEG_CHUNKS, edge_loop, None)
    plsc.subcore_barrier()
    pltpu.sync_copy(deg_sp.at[pl.ds(r0, _RPT)], buf_v)
    pltpu.sync_copy(buf_v, degp_hbm.at[c, pl.ds(r0, _RPT)])


@functools.partial(
    pl.kernel,
    out_type=(
        jax.ShapeDtypeStruct((2, _NPAD, _HALF), jnp.float32),
        jax.ShapeDtypeStruct((2, _NPAD, _HALF), jnp.float32),
    ),
    mesh=_mesh,
    compiler_params=_cp,
    scratch_types=[
        pltpu.VMEM((_PROP_CHUNKS, _CHUNK), jnp.int32),
        pltpu.VMEM((_PROP_CHUNKS, _CHUNK), jnp.int32),
        pltpu.VMEM((_CHUNK, _HALF), jnp.float32),
        pltpu.VMEM((_RPT, _HALF), jnp.float32),
        pltpu.VMEM((_RPT,), jnp.float32),
        pltpu.VMEM_SHARED((_NPAD, _HALF), jnp.float32),
    ],
)
def _prop_kernel(z0, esrc, edst, dinv_hbm, u2, z1,
                 src_v, dst_v, rows_v, rowbuf, dinv_v, acc_sp):
    c = lax.axis_index("c")
    s = lax.axis_index("s")
    r0 = s * _RPT
    tbl0 = z0.at[c]
    tbl1 = z1.at[c]
    pltpu.sync_copy(esrc.at[s], src_v)
    pltpu.sync_copy(edst.at[s], dst_v)
    pltpu.sync_copy(dinv_hbm.at[pl.ds(r0, _RPT)], dinv_v)
    pltpu.sync_copy(tbl0.at[pl.ds(r0, _RPT)], rowbuf)
    pltpu.sync_copy(rowbuf, acc_sp.at[pl.ds(r0, _RPT)])
    plsc.subcore_barrier()

    def hop(table, i, _):
        pltpu.sync_copy(table.at[src_v.at[i]], rows_v)
        pltpu.sync_copy(rows_v, acc_sp.at[dst_v.at[i]], add=True)
        return _

    lax.fori_loop(0, _PROP_CHUNKS, functools.partial(hop, tbl0), None)
    plsc.subcore_barrier()

    pltpu.sync_copy(acc_sp.at[pl.ds(r0, _RPT)], rowbuf)

    def scale(r, _):
        dscale = plsc.load_gather(dinv_v, [jnp.full((16,), r, jnp.int32)])
        for h in range(_HALF // 16):
            v = rowbuf[r, pl.ds(h * 16, 16)]
            rowbuf[r, pl.ds(h * 16, 16)] = v * dscale
        return _
    lax.fori_loop(0, _RPT, scale, None)

    pltpu.sync_copy(rowbuf, tbl1.at[pl.ds(r0, _RPT)])
    pltpu.sync_copy(rowbuf, acc_sp.at[pl.ds(r0, _RPT)])
    plsc.subcore_barrier()

    lax.fori_loop(0, _PROP_CHUNKS, functools.partial(hop, tbl1), None)
    plsc.subcore_barrier()

    pltpu.sync_copy(acc_sp.at[pl.ds(r0, _RPT)], rowbuf)
    pltpu.sync_copy(rowbuf, u2.at[c, pl.ds(r0, _RPT)])


def _tc_pre_body(x_ref, w_ref, degp_ref, z0_ref, dinv_ref, dis_ref):
    deg = degp_ref[0, :] + degp_ref[1, :]
    dis = lax.rsqrt(deg)
    y = lax.dot_general(x_ref[...], w_ref[...], (((1,), (1,)), ((), ())),
                        preferred_element_type=jnp.float32)
    z = y * dis[:, None]
    z0_ref[0] = z[:, :_HALF]
    z0_ref[1] = z[:, _HALF:]
    dinv_ref[...] = 1.0 / deg
    dis_ref[...] = dis


def _tc_pre(x_pad, w, degp):
    blk = 256
    grid = _NPAD // blk
    return pl.pallas_call(
        _tc_pre_body,
        grid=(grid,),
        in_specs=[
            pl.BlockSpec((blk, _D), lambda i: (i, 0)),
            pl.BlockSpec((_C, _D), lambda i: (0, 0)),
            pl.BlockSpec((2, blk), lambda i: (0, i)),
        ],
        out_specs=[
            pl.BlockSpec((2, blk, _HALF), lambda i: (0, i, 0)),
            pl.BlockSpec((blk,), lambda i: (i,)),
            pl.BlockSpec((blk,), lambda i: (i,)),
        ],
        out_shape=[
            jax.ShapeDtypeStruct((2, _NPAD, _HALF), jnp.float32),
            jax.ShapeDtypeStruct((_NPAD,), jnp.float32),
            jax.ShapeDtypeStruct((_NPAD,), jnp.float32),
        ],
    )(x_pad, w, degp)


def _tc_post_body(u2_ref, dis_ref, b_ref, out_ref):
    h = jnp.concatenate([u2_ref[0], u2_ref[1]], axis=1)
    logits = h * dis_ref[...][:, None] + b_ref[...][None, :]
    m = jnp.max(logits, axis=1, keepdims=True)
    shifted = logits - m
    lse = jnp.log(jnp.sum(jnp.exp(shifted), axis=1, keepdims=True))
    out_ref[...] = shifted - lse


def _tc_post(u2, dis, b):
    blk = 256
    grid = _NPAD // blk
    return pl.pallas_call(
        _tc_post_body,
        grid=(grid,),
        in_specs=[
            pl.BlockSpec((2, blk, _HALF), lambda i: (0, i, 0)),
            pl.BlockSpec((blk,), lambda i: (i,)),
            pl.BlockSpec((_C,), lambda i: (0,)),
        ],
        out_specs=pl.BlockSpec((blk, _C), lambda i: (i, 0)),
        out_shape=jax.ShapeDtypeStruct((_NPAD, _C), jnp.float32),
    )(u2, dis, b)


def kernel(x, edge_index, W, b):
    x_pad = jnp.zeros((_NPAD, _D), jnp.float32).at[:_N].set(x)
    src = edge_index[0].reshape(_NSUB, _PROP_CHUNKS, _CHUNK)
    dst = edge_index[1].reshape(_NSUB, _PROP_CHUNKS, _CHUNK)
    dst32 = edge_index[1].reshape(32, _DEG_CHUNKS, _CHUNK)

    degp = _deg_kernel(dst32)
    z0, dinv, dis = _tc_pre(x_pad, W, degp)
    u2, _ = _prop_kernel(z0, src, dst, dinv)
    out = _tc_post(u2, dis, b)
    return out[:_N]
